# Initial kernel scaffold; baseline (speedup 1.0000x reference)
import jax
import jax.numpy as jnp
from jax.experimental import pallas as pl


def kernel(eeg_embeddings, clip_embeddings, queue, random_indices):
    raise NotImplementedError("write your pallas kernel here")


# speedup vs baseline: 12.3614x; 12.3614x over previous
import functools

import jax
import jax.numpy as jnp
from jax import lax
from jax.experimental import pallas as pl
from jax.experimental.pallas import tpu as pltpu
from jax.experimental.pallas import tpu_sc as plsc

_B = 4096
_D = 512
_Q = 65536
_K = 19660
_NR = _Q - _K
_INV_T = float(1.0 / 0.07)


_RB = 256
_QB = 1024


def _mm_body(eeg_ref, clip_ref, q_ref, neg_ref, pos_ref):
    e = eeg_ref[:]
    en = e * lax.rsqrt(jnp.sum(e * e, axis=1, keepdims=True))
    c = clip_ref[:]
    cn = c * lax.rsqrt(jnp.sum(c * c, axis=1, keepdims=True))
    pos_ref[:] = jnp.sum(en * cn, axis=1, keepdims=True)
    neg_ref[:] = lax.dot_general(en, q_ref[:], (((1,), (1,)), ((), ())),
                                 preferred_element_type=jnp.float32)


_phase1 = pl.pallas_call(
    _mm_body,
    grid=(_Q // _QB, _B // _RB),
    in_specs=[
        pl.BlockSpec((_RB, _D), lambda q, b: (b, 0)),
        pl.BlockSpec((_RB, _D), lambda q, b: (b, 0)),
        pl.BlockSpec((_QB, _D), lambda q, b: (q, 0)),
    ],
    out_specs=[
        pl.BlockSpec((_RB, _QB), lambda q, b: (b, q)),
        pl.BlockSpec((_RB, 1), lambda q, b: (b, 0)),
    ],
    out_shape=[
        jax.ShapeDtypeStruct((_B, _Q), jnp.float32),
        jax.ShapeDtypeStruct((_B, 1), jnp.float32),
    ],
)


_R2 = 32
_NCH = 16
_CH = _Q // _NCH
_BISECT = 14


def _stats_body(neg_ref, m_ref, sh_ref):
    def maxstep(j, acc):
        return jnp.maximum(acc, jnp.max(neg_ref[:, j, :], axis=1, keepdims=True))

    m = lax.fori_loop(0, _NCH, maxstep,
                      jnp.full((_R2, 1), -2.0, jnp.float32))

    def bis(i, lohi):
        lo, hi = lohi
        mid = 0.5 * (lo + hi)

        def cstep(j, cnt):
            ch = neg_ref[:, j, :]
            return cnt + jnp.sum((ch > mid).astype(jnp.float32), axis=1,
                                 keepdims=True)

        cnt = lax.fori_loop(0, _NCH, cstep, jnp.zeros((_R2, 1), jnp.float32))
        ge = cnt >= float(_K)
        return (jnp.where(ge, mid, lo), jnp.where(ge, hi, mid))

    lo0 = jnp.full((_R2, 1), -1.01, jnp.float32)
    hi0 = jnp.full((_R2, 1), 1.01, jnp.float32)
    _, hi = lax.fori_loop(0, _BISECT, bis, (lo0, hi0))

    def fstep(j, sc):
        s, c = sc
        ch = neg_ref[:, j, :]
        ab = ch > hi
        e = jnp.exp(ch * _INV_T)
        s = s + jnp.sum(jnp.where(ab, e, 0.0), axis=1, keepdims=True)
        c = c + jnp.sum(ab.astype(jnp.float32), axis=1, keepdims=True)
        return (s, c)

    s, c = lax.fori_loop(0, _NCH, fstep,
                         (jnp.zeros((_R2, 1), jnp.float32),
                          jnp.zeros((_R2, 1), jnp.float32)))
    m_ref[:] = m
    sh_ref[:] = s + (float(_K) - c) * jnp.exp(hi * _INV_T)


_phase2 = pl.pallas_call(
    _stats_body,
    grid=(_B // _R2,),
    in_specs=[pl.BlockSpec((_R2, _NCH, _CH), lambda i: (i, 0, 0))],
    out_specs=[
        pl.BlockSpec((_R2, 1), lambda i: (i, 0)),
        pl.BlockSpec((_R2, 1), lambda i: (i, 0)),
    ],
    out_shape=[
        jax.ShapeDtypeStruct((_B, 1), jnp.float32),
        jax.ShapeDtypeStruct((_B, 1), jnp.float32),
    ],
)


_NW = 32
_RPW = _B // _NW
_NRP = 45888
_NVEC = _NRP // 16

_sc_mesh = plsc.VectorSubcoreMesh(core_axis_name="c", subcore_axis_name="s")


@functools.partial(
    pl.kernel,
    mesh=_sc_mesh,
    out_type=jax.ShapeDtypeStruct((_B,), jnp.float32),
    scratch_types=[
        pltpu.VMEM((_Q,), jnp.float32),
        pltpu.VMEM((_NRP,), jnp.int32),
        pltpu.VMEM((_RPW,), jnp.float32),
    ],
    compiler_params=pltpu.CompilerParams(needs_layout_passes=False),
)
def _phase3(neg_hbm, ridx_hbm, out_hbm, row_v, idx_v, res_v):
    wid = lax.axis_index("s") * 2 + lax.axis_index("c")
    base = wid * _RPW
    lane = lax.iota(jnp.int32, 16)

    def group_body(grp, carry):
        def row_body(rr, res):
            row = base + grp * 16 + rr
            pltpu.sync_copy(ridx_hbm.at[row], idx_v)
            pltpu.sync_copy(neg_hbm.at[row], row_v)

            def gstep(i, acc):
                idx = idx_v[pl.ds(i * 16, 16)]
                g = plsc.load_gather(row_v, [idx])
                return acc + jnp.exp(g * _INV_T)

            acc = lax.fori_loop(0, _NVEC, gstep, jnp.zeros((16,), jnp.float32),
                                unroll=8)
            v0 = row_v[pl.ds(0, 16)]
            e0 = jnp.exp(v0 * _INV_T)
            acc = acc - 12.0 * jnp.where(lane == 0, e0, 0.0)
            return jnp.where(lane == rr, jnp.sum(acc), res)

        res = lax.fori_loop(0, 16, row_body, jnp.zeros((16,), jnp.float32))
        res_v[pl.ds(grp * 16, 16)] = res
        return carry

    lax.fori_loop(0, _RPW // 16, group_body, 0)
    pltpu.sync_copy(res_v, out_hbm.at[pl.ds(base, _RPW)])


def _final_body(pos_ref, m_ref, sh_ref, sr_ref, loss_ref, acc_ref):
    pos = pos_ref[:]
    pt = pos * _INV_T
    lse = jnp.log(jnp.exp(pt) + sh_ref[:] + sr_ref[:])
    loss_ref[:, :] = jnp.mean(lse - pt, keepdims=True)
    acc_ref[:, :] = jnp.mean((pos >= m_ref[:]).astype(jnp.float32),
                             keepdims=True)


_phase4 = pl.pallas_call(
    _final_body,
    out_shape=[
        jax.ShapeDtypeStruct((1, 1), jnp.float32),
        jax.ShapeDtypeStruct((1, 1), jnp.float32),
    ],
)


def kernel(eeg_embeddings, clip_embeddings, queue, random_indices):
    neg, pos = _phase1(eeg_embeddings, clip_embeddings, queue)
    m, sh = _phase2(neg.reshape(_B, _NCH, _CH))
    ridx_p = jnp.pad(random_indices.astype(jnp.int32), ((0, 0), (0, _NRP - _NR)))
    sr = _phase3(neg, ridx_p)
    loss, acc = _phase4(pos.reshape(32, 128), m.reshape(32, 128),
                        sh.reshape(32, 128), sr.reshape(32, 128))
    return loss.reshape(()), acc.reshape(())
